# baseline (speedup 1.0000x reference)
import functools
import jax
import jax.numpy as jnp
from jax import lax
from jax.experimental import pallas as pl
from jax.experimental.pallas import tpu as pltpu
from jax.experimental.pallas import tpu_sc as plsc

_EPS = 1e-5
_MAX_NB = 64


def _fps_body(n_pts, n_samp, px_ref, py_ref, pz_ref, sel_ref, qx_ref, qy_ref,
              qz_ref):
    px = px_ref[...]
    py = py_ref[...]
    pz = pz_ref[...]
    nb = px.shape[0]
    lane = jax.lax.broadcasted_iota(jnp.int32, (nb, n_pts), 1)
    lane_q = jax.lax.broadcasted_iota(jnp.int32, (nb, n_samp), 1)

    def step(i, carry):
        last, dist, sel, qx, qy, qz = carry
        is_last = lane == last
        lx = jnp.sum(jnp.where(is_last, px, 0.0), axis=1, keepdims=True)
        ly = jnp.sum(jnp.where(is_last, py, 0.0), axis=1, keepdims=True)
        lz = jnp.sum(jnp.where(is_last, pz, 0.0), axis=1, keepdims=True)
        rec = lane_q == i
        qx = jnp.where(rec, lx, qx)
        qy = jnp.where(rec, ly, qy)
        qz = jnp.where(rec, lz, qz)
        d = (px - lx) ** 2 + (py - ly) ** 2 + (pz - lz) ** 2
        dist = jnp.minimum(dist, d)
        m = jnp.max(dist, axis=1, keepdims=True)
        nxt = jnp.min(jnp.where(dist >= m, lane, 2 ** 30), axis=1,
                      keepdims=True)
        sel = jnp.where(lane_q == i + 1, nxt, sel)
        return nxt, dist, sel, qx, qy, qz

    init = (jnp.zeros((nb, 1), jnp.int32),
            jnp.full((nb, n_pts), jnp.inf, jnp.float32),
            jnp.zeros((nb, n_samp), jnp.int32),
            jnp.zeros((nb, n_samp), jnp.float32),
            jnp.zeros((nb, n_samp), jnp.float32),
            jnp.zeros((nb, n_samp), jnp.float32))
    _, _, sel, qx, qy, qz = jax.lax.fori_loop(0, n_samp, step, init)
    sel_ref[...] = sel
    qx_ref[...] = qx
    qy_ref[...] = qy
    qz_ref[...] = qz


def _fps(px, py, pz, n_samp):
    nb, n_pts = px.shape
    out = (jax.ShapeDtypeStruct((nb, n_samp), jnp.int32),
           jax.ShapeDtypeStruct((nb, n_samp), jnp.float32),
           jax.ShapeDtypeStruct((nb, n_samp), jnp.float32),
           jax.ShapeDtypeStruct((nb, n_samp), jnp.float32))
    return pl.pallas_call(
        functools.partial(_fps_body, n_pts, n_samp),
        out_shape=out,
    )(px, py, pz)


def _u_body(x_ref, p_ref, wx_ref, wp_ref, o_ref):
    x = x_ref[...]
    p = p_ref[...]
    u = jnp.dot(x, wx_ref[...], preferred_element_type=jnp.float32)
    u += jnp.dot(p, wp_ref[...], preferred_element_type=jnp.float32)
    o_ref[...] = u


def _compute_u(x2d, p2d, wx, wp):
    n = x2d.shape[0]
    d = wx.shape[1]
    return pl.pallas_call(
        _u_body,
        out_shape=jax.ShapeDtypeStruct((n, d), jnp.float32),
    )(x2d, p2d, wx, wp)


def _max_mid(m3):
    k = m3.shape[1]
    while k > 1:
        h = k // 2
        m3 = jnp.maximum(m3[:, :h, :], m3[:, h:, :])
        k = h
    return m3[:, 0, :]


def _pair_body(qb, d1, g_ref, posq_ref, maskf_ref, wp_ref, b1_ref, w2_ref,
               b2_ref, w3_ref, b3_ref, o_ref):
    k = _MAX_NB
    dpad = g_ref.shape[2]
    posq = posq_ref[0]
    c = b1_ref[...] - jnp.dot(posq, wp_ref[...],
                              preferred_element_type=jnp.float32)
    g = g_ref[0].reshape(qb, k, dpad)
    if dpad != d1:
        g = g[:, :, :d1]
    a1 = jnp.maximum(g + c[:, None, :], 0.0)
    a1 = a1.reshape(qb * k, d1)
    a2 = jnp.dot(a1, w2_ref[...], preferred_element_type=jnp.float32)
    a2 = jnp.maximum(a2 + b2_ref[...], 0.0)
    m = jnp.dot(a2, w3_ref[...], preferred_element_type=jnp.float32)
    m = m + b3_ref[...]
    dout = m.shape[1]
    m = m.reshape(qb, k, dout) + maskf_ref[0][:, :, None]
    o_ref[0] = _max_mid(m)


def _pair_mlp(g, posq, maskf, wp, b1, w2, b2, w3, b3, qb):
    nb, q, _ = posq.shape
    k = _MAX_NB
    dpad = g.shape[2]
    d1 = w2.shape[0]
    dh = w2.shape[1]
    dout = w3.shape[1]
    grid = (nb, q // qb)
    return pl.pallas_call(
        functools.partial(_pair_body, qb, d1),
        grid=grid,
        in_specs=[
            pl.BlockSpec((1, qb * k, dpad), lambda b, t: (b, t, 0)),
            pl.BlockSpec((1, qb, 3), lambda b, t: (b, t, 0)),
            pl.BlockSpec((1, qb, k), lambda b, t: (b, t, 0)),
            pl.BlockSpec((3, d1), lambda b, t: (0, 0)),
            pl.BlockSpec((1, d1), lambda b, t: (0, 0)),
            pl.BlockSpec((d1, dh), lambda b, t: (0, 0)),
            pl.BlockSpec((1, dh), lambda b, t: (0, 0)),
            pl.BlockSpec((dh, dout), lambda b, t: (0, 0)),
            pl.BlockSpec((1, dout), lambda b, t: (0, 0)),
        ],
        out_specs=pl.BlockSpec((1, qb, dout), lambda b, t: (b, t, 0)),
        out_shape=jax.ShapeDtypeStruct((nb, q, dout), jnp.float32),
    )(g, posq, maskf, wp, b1, w2, b2, w3, b3)


def _sa3_body(n_cloud, q, x_ref, p_ref, w0x_ref, w0p_ref, b0_ref, w1_ref,
              b1_ref, w2_ref, b2_ref, h0_ref, hb0_ref, h1_ref, hb1_ref,
              h2_ref, hb2_ref, o_ref):
    x = x_ref[...]
    p = p_ref[...]
    h = jnp.dot(x, w0x_ref[...], preferred_element_type=jnp.float32)
    h += jnp.dot(p, w0p_ref[...], preferred_element_type=jnp.float32)
    h = jnp.maximum(h + b0_ref[...], 0.0)
    h = jnp.dot(h, w1_ref[...], preferred_element_type=jnp.float32)
    h = jnp.maximum(h + b1_ref[...], 0.0)
    h = jnp.dot(h, w2_ref[...], preferred_element_type=jnp.float32)
    h = h + b2_ref[...]
    h = h.reshape(n_cloud, q, h.shape[1])
    g = _max_mid(h)
    g = jnp.dot(g, h0_ref[...], preferred_element_type=jnp.float32)
    g = jnp.maximum(g + hb0_ref[...], 0.0)
    g = jnp.dot(g, h1_ref[...], preferred_element_type=jnp.float32)
    g = jnp.maximum(g + hb1_ref[...], 0.0)
    g = jnp.dot(g, h2_ref[...], preferred_element_type=jnp.float32)
    g = g + hb2_ref[...]
    mx = jnp.max(g, axis=1, keepdims=True)
    lse = jnp.log(jnp.sum(jnp.exp(g - mx), axis=1, keepdims=True)) + mx
    o_ref[...] = g - lse


def _sa3_head(x2d, p2d, n_cloud, q, args):
    ncls = args[-1].shape[1]
    return pl.pallas_call(
        functools.partial(_sa3_body, n_cloud, q),
        out_shape=jax.ShapeDtypeStruct((n_cloud, ncls), jnp.float32),
    )(x2d, p2d, *args)


def _sel_body(r2, k, px_ref, py_ref, pz_ref, qx_ref, qy_ref, qz_ref,
              pcat_ref, words_ref, cnt_ref):
    pxc = px_ref[0]
    pyc = py_ref[0]
    pzc = pz_ref[0]
    qxr = qx_ref[0]
    qyr = qy_ref[0]
    qzr = qz_ref[0]
    d2 = (pxc - qxr) ** 2 + (pyc - qyr) ** 2 + (pzc - qzr) ** 2
    bits = jax.lax.bitcast_convert_type(d2, jnp.int32)
    big = 0x7F000000
    valid = d2 <= r2
    v = jnp.where(valid, bits, big)
    p = jnp.zeros((1, v.shape[1]), jnp.int32)
    for b in range(29, -1, -1):
        t = p + ((1 << b) - 1)
        cnt = jnp.sum((v <= t).astype(jnp.int32), axis=0, keepdims=True)
        p = jnp.where(cnt < k, p + (1 << b), p)
    sel = (v <= p) & valid
    cnt_ref[0] = jnp.sum(sel.astype(jnp.int32), axis=0, keepdims=True)
    w2 = jnp.dot(pcat_ref[...], jnp.where(sel, 1.0, 0.0),
                 preferred_element_type=jnp.float32)
    wi = w2.astype(jnp.int32)
    nww = wi.shape[0] // 2
    words_ref[0] = wi[:nww] | (wi[nww:] << 16)


def _select_words(posq, pb, r):
    nb, q, _ = posq.shape
    n = pb.shape[1]
    nww = n // 32
    px = pb[:, :, 0:1]
    py = pb[:, :, 1:2]
    pz = pb[:, :, 2:3]
    qx = posq[:, :, 0][:, None, :]
    qy = posq[:, :, 1][:, None, :]
    qz = posq[:, :, 2][:, None, :]
    jj = jnp.arange(n, dtype=jnp.int32)
    ww = jnp.arange(nww, dtype=jnp.int32)
    rel = jj[None, :] - ww[:, None] * 32
    plo = jnp.where((rel >= 0) & (rel < 16), 2.0 ** rel, 0.0)
    phi = jnp.where((rel >= 16) & (rel < 32), 2.0 ** (rel - 16), 0.0)
    pcat = jnp.concatenate([plo, phi], axis=0).astype(jnp.float32)
    words3, cnt3 = pl.pallas_call(
        functools.partial(_sel_body, r * r, _MAX_NB),
        grid=(nb,),
        in_specs=[
            pl.BlockSpec((1, n, 1), lambda b: (b, 0, 0)),
            pl.BlockSpec((1, n, 1), lambda b: (b, 0, 0)),
            pl.BlockSpec((1, n, 1), lambda b: (b, 0, 0)),
            pl.BlockSpec((1, 1, q), lambda b: (b, 0, 0)),
            pl.BlockSpec((1, 1, q), lambda b: (b, 0, 0)),
            pl.BlockSpec((1, 1, q), lambda b: (b, 0, 0)),
            pl.BlockSpec((2 * nww, n), lambda b: (0, 0)),
        ],
        out_specs=[
            pl.BlockSpec((1, nww, q), lambda b: (b, 0, 0)),
            pl.BlockSpec((1, 1, q), lambda b: (b, 0, 0)),
        ],
        out_shape=[
            jax.ShapeDtypeStruct((nb, nww, q), jnp.int32),
            jax.ShapeDtypeStruct((nb, 1, q), jnp.int32),
        ],
    )(px, py, pz, qx, qy, qz, pcat)
    return words3, cnt3[:, 0, :]


def _sc_select_gather(words_flat, cnt_flat, u_flat, nb, n, q, dpad):
    k = _MAX_NB
    nww = n // 32
    info = plsc.get_sparse_core_info()
    nw = info.num_cores * info.num_subcores
    tq = nb * q
    qpw = tq // nw
    mesh = plsc.VectorSubcoreMesh(core_axis_name="c", subcore_axis_name="s")

    gb = 8

    @functools.partial(
        pl.kernel, mesh=mesh,
        out_type=jax.ShapeDtypeStruct((tq * k, dpad), jnp.float32),
        scratch_types=[
            pltpu.VMEM((qpw * nww + 16,), jnp.int32),
            pltpu.VMEM((qpw + 16,), jnp.int32),
            pltpu.VMEM((gb * k,), jnp.int32),
            pltpu.VMEM((gb * k, dpad), jnp.float32),
            pltpu.SemaphoreType.DMA,
        ],
    )
    def sel_gather_k(w_hbm, cnt_hbm, u_hbm, out_hbm, wv, cntv, idx64,
                     pairs_v, sem):
        wid = lax.axis_index("s") * info.num_cores + lax.axis_index("c")
        qoff = wid * qpw
        cloud = qoff // q
        jbase = cloud * n
        pltpu.sync_copy(w_hbm.at[pl.ds(qoff * nww, qpw * nww)],
                        wv.at[pl.ds(0, qpw * nww)])
        pltpu.sync_copy(cnt_hbm.at[pl.ds(qoff, qpw)],
                        cntv.at[pl.ds(0, qpw)])
        zeros16 = jnp.zeros((16,), jnp.int32)
        lane = lax.iota(jnp.int32, 16)

        def scan_query(qi, slot):
            w0 = wv[pl.ds(qi * nww, 16)][0]
            cq = cntv[pl.ds(qi, 16)][0]
            trip = jnp.minimum(cq, k) + nww

            def bit_body(s, st):
                wi, w, off, b0, b1, b2, b3 = st
                active = w != 0
                lsb = w & (-w)
                fb = jax.lax.bitcast_convert_type(
                    lsb.astype(jnp.float32), jnp.int32)
                p = jnp.where(lsb > 0, (fb >> 23) - 127, 31)
                j = jbase + wi * 32 + p
                ins = active & (off < k)
                bk = off >> 4
                pp = off & 15
                p0 = jnp.where(ins & (bk == 0), pp, 16)
                p1 = jnp.where(ins & (bk == 1), pp, 16)
                p2 = jnp.where(ins & (bk == 2), pp, 16)
                p3 = jnp.where(ins & (bk == 3), pp, 16)
                b0 = jnp.where(lane == p0, j, b0)
                b1 = jnp.where(lane == p1, j, b1)
                b2 = jnp.where(lane == p2, j, b2)
                b3 = jnp.where(lane == p3, j, b3)
                nwi = jnp.where(active, wi, wi + 1)
                nxt = wv[pl.ds(qi * nww + jnp.minimum(nwi, nww - 1), 16)][0]
                nw_ = jnp.where(active, w & (w - 1),
                                jnp.where(nwi < nww, nxt, 0))
                return (nwi, nw_, off + jnp.where(ins, 1, 0), b0, b1, b2, b3)

            st = lax.fori_loop(0, trip, bit_body,
                               (0, w0, 0, zeros16 + jbase, zeros16 + jbase,
                                zeros16 + jbase, zeros16 + jbase))
            idx64[pl.ds(slot * k, 16)] = st[3]
            idx64[pl.ds(slot * k + 16, 16)] = st[4]
            idx64[pl.ds(slot * k + 32, 16)] = st[5]
            idx64[pl.ds(slot * k + 48, 16)] = st[6]

        def g_body(gi, carry):
            q0 = gi * gb
            for t in range(gb):
                scan_query(q0 + t, t)
            pltpu.async_copy(u_hbm.at[idx64], pairs_v, sem).wait()
            pltpu.sync_copy(pairs_v,
                            out_hbm.at[pl.ds((qoff + q0) * k, gb * k)])
            return carry

        lax.fori_loop(0, qpw // gb, g_body, 0)

    return sel_gather_k(words_flat, cnt_flat, u_flat)


def _fold_norm(params):
    s = (1.0 + _EPS) ** 0.5
    ws, bs = [], []
    n = len(params["W"])
    for i in range(n):
        w, b = params["W"][i], params["b"][i]
        if i < n - 1:
            g, beta = params["g"][i], params["beta"][i]
            ws.append(w * (g / s)[None, :])
            bs.append(b * g / s + beta)
        else:
            ws.append(w)
            bs.append(b)
    return ws, bs


def _sc_gather(u_flat, idx_flat, d):
    tot = idx_flat.shape[0]
    info = plsc.get_sparse_core_info()
    nw = info.num_cores * info.num_subcores
    rows_w = tot // nw
    ch = 512
    n_chunk = rows_w // ch
    mesh = plsc.VectorSubcoreMesh(core_axis_name="c", subcore_axis_name="s")

    @functools.partial(
        pl.kernel, mesh=mesh,
        out_type=jax.ShapeDtypeStruct((tot, d), jnp.float32),
        scratch_types=[
            pltpu.VMEM((ch,), jnp.int32),
            pltpu.VMEM((ch, d), jnp.float32),
            pltpu.SemaphoreType.DMA,
        ],
    )
    def gather_k(u_hbm, idx_hbm, out_hbm, idx_v, rows_v, sem):
        wid = lax.axis_index("s") * info.num_cores + lax.axis_index("c")
        base = wid * rows_w

        def chunk(ci, carry):
            off = base + ci * ch
            pltpu.sync_copy(idx_hbm.at[pl.ds(off, ch)], idx_v)
            pltpu.async_copy(u_hbm.at[idx_v], rows_v, sem).wait()
            pltpu.sync_copy(rows_v, out_hbm.at[pl.ds(off, ch)])
            return carry

        lax.fori_loop(0, n_chunk, chunk, 0)

    return gather_k(u_flat, idx_flat)


def _gather_pairs(u, idx):
    nb, q, k = idx.shape
    n, d = u.shape[1], u.shape[2]
    gbase = jnp.arange(nb, dtype=jnp.int32)[:, None, None] * n
    flat = (idx + gbase).reshape(nb * q * k)
    out = _sc_gather(u.reshape(nb * n, d), flat, d)
    return out.reshape(nb, q * k, d)


def _sa_stage(xb, posb, px, py, pz, params, ratio, r, qb):
    nb, n_pts, fdim = xb.shape
    n_samp = int(n_pts * ratio)
    ws, bs = _fold_norm(params)
    w1, w2, w3 = ws
    b1, b2, b3 = bs
    d1 = w1.shape[1]

    sel, qx, qy, qz = _fps(px, py, pz, n_samp)
    posq = jnp.stack([qx, qy, qz], axis=-1)

    u = _compute_u(xb.reshape(nb * n_pts, fdim),
                   posb.reshape(nb * n_pts, 3),
                   w1[:fdim], w1[fdim:]).reshape(nb, n_pts, d1)
    if d1 < 128:
        u = jnp.pad(u, ((0, 0), (0, 0), (0, 128 - d1)))
    dpad = u.shape[2]

    words3, cnt = _select_words(posq, posb, r)
    words_flat = jnp.transpose(words3, (0, 2, 1)).reshape(-1)
    pairs = _sc_select_gather(words_flat, cnt.reshape(-1),
                              u.reshape(nb * n_pts, dpad), nb, n_pts,
                              n_samp, dpad)
    g = pairs.reshape(nb, n_samp * _MAX_NB, dpad)
    ik = jnp.arange(_MAX_NB, dtype=jnp.int32)[None, None, :]
    maskf = jnp.where(ik < jnp.minimum(cnt, _MAX_NB)[:, :, None], 0.0,
                      -jnp.inf).astype(jnp.float32)

    x_out = _pair_mlp(g, posq, maskf, w1[fdim:], b1[None, :], w2,
                      b2[None, :], w3, b3[None, :], qb)
    return x_out, posq, qx, qy, qz


def kernel(x, pos, batch, params):
    nb = batch.shape[0] // 1024
    n_pts = 1024
    xb = x.reshape(nb, n_pts, -1)
    pb = pos.reshape(nb, n_pts, 3)
    px = pb[:, :, 0]
    py = pb[:, :, 1]
    pz = pb[:, :, 2]

    x1, posq1, q1x, q1y, q1z = _sa_stage(xb, pb, px, py, pz, params["sa1"],
                                         0.5, 0.2, 128)
    x2, posq2, _, _, _ = _sa_stage(x1, posq1, q1x, q1y, q1z, params["sa2"],
                                   0.25, 0.4, 128)

    ws, bs = _fold_norm(params["sa3"])
    hw, hb = _fold_norm(params["head"])
    q2 = x2.shape[1]
    f2 = x2.shape[2]
    args = (ws[0][:f2], ws[0][f2:], bs[0][None, :], ws[1], bs[1][None, :],
            ws[2], bs[2][None, :], hw[0], hb[0][None, :], hw[1],
            hb[1][None, :], hw[2], hb[2][None, :])
    return _sa3_head(x2.reshape(nb * q2, f2), posq2.reshape(nb * q2, 3),
                     nb, q2, args)


# speedup vs baseline: 1.3522x; 1.3522x over previous
import functools
import jax
import jax.numpy as jnp
from jax import lax
from jax.experimental import pallas as pl
from jax.experimental.pallas import tpu as pltpu
from jax.experimental.pallas import tpu_sc as plsc

_EPS = 1e-5
_MAX_NB = 64


def _fps_body(n_pts, n_samp, px_ref, py_ref, pz_ref, sel_ref, qx_ref, qy_ref,
              qz_ref):
    px = px_ref[...]
    py = py_ref[...]
    pz = pz_ref[...]
    nb = px.shape[0]
    lane = jax.lax.broadcasted_iota(jnp.int32, (nb, n_pts), 1)
    lane_q = jax.lax.broadcasted_iota(jnp.int32, (nb, n_samp), 1)

    def step(i, carry):
        last, dist, sel, qx, qy, qz = carry
        is_last = lane == last
        lx = jnp.sum(jnp.where(is_last, px, 0.0), axis=1, keepdims=True)
        ly = jnp.sum(jnp.where(is_last, py, 0.0), axis=1, keepdims=True)
        lz = jnp.sum(jnp.where(is_last, pz, 0.0), axis=1, keepdims=True)
        rec = lane_q == i
        qx = jnp.where(rec, lx, qx)
        qy = jnp.where(rec, ly, qy)
        qz = jnp.where(rec, lz, qz)
        d = (px - lx) ** 2 + (py - ly) ** 2 + (pz - lz) ** 2
        dist = jnp.minimum(dist, d)
        m = jnp.max(dist, axis=1, keepdims=True)
        nxt = jnp.min(jnp.where(dist >= m, lane, 2 ** 30), axis=1,
                      keepdims=True)
        sel = jnp.where(lane_q == i + 1, nxt, sel)
        return nxt, dist, sel, qx, qy, qz

    init = (jnp.zeros((nb, 1), jnp.int32),
            jnp.full((nb, n_pts), jnp.inf, jnp.float32),
            jnp.zeros((nb, n_samp), jnp.int32),
            jnp.zeros((nb, n_samp), jnp.float32),
            jnp.zeros((nb, n_samp), jnp.float32),
            jnp.zeros((nb, n_samp), jnp.float32))
    _, _, sel, qx, qy, qz = jax.lax.fori_loop(0, n_samp, step, init)
    sel_ref[...] = sel
    qx_ref[...] = qx
    qy_ref[...] = qy
    qz_ref[...] = qz


def _fps(px, py, pz, n_samp):
    nb, n_pts = px.shape
    out = (jax.ShapeDtypeStruct((nb, n_samp), jnp.int32),
           jax.ShapeDtypeStruct((nb, n_samp), jnp.float32),
           jax.ShapeDtypeStruct((nb, n_samp), jnp.float32),
           jax.ShapeDtypeStruct((nb, n_samp), jnp.float32))
    return pl.pallas_call(
        functools.partial(_fps_body, n_pts, n_samp),
        out_shape=out,
    )(px, py, pz)


def _u_body(x_ref, p_ref, wx_ref, wp_ref, o_ref):
    x = x_ref[...]
    p = p_ref[...]
    u = jnp.dot(x, wx_ref[...], preferred_element_type=jnp.float32)
    u += jnp.dot(p, wp_ref[...], preferred_element_type=jnp.float32)
    o_ref[...] = u


def _compute_u(x2d, p2d, wx, wp):
    n = x2d.shape[0]
    d = wx.shape[1]
    return pl.pallas_call(
        _u_body,
        out_shape=jax.ShapeDtypeStruct((n, d), jnp.float32),
    )(x2d, p2d, wx, wp)


def _max_mid(m3):
    k = m3.shape[1]
    while k > 1:
        h = k // 2
        m3 = jnp.maximum(m3[:, :h, :], m3[:, h:, :])
        k = h
    return m3[:, 0, :]


def _pair_body(qb, d1, g_ref, posq_ref, maskf_ref, wp_ref, b1_ref, w2_ref,
               b2_ref, w3_ref, b3_ref, o_ref):
    k = _MAX_NB
    dpad = g_ref.shape[2]
    posq = posq_ref[0]
    c = b1_ref[...] - jnp.dot(posq, wp_ref[...],
                              preferred_element_type=jnp.float32)
    g = g_ref[0].reshape(qb, k, dpad)
    if dpad != d1:
        g = g[:, :, :d1]
    a1 = jnp.maximum(g + c[:, None, :], 0.0)
    a1 = a1.reshape(qb * k, d1)
    a2 = jnp.dot(a1, w2_ref[...], preferred_element_type=jnp.float32)
    a2 = jnp.maximum(a2 + b2_ref[...], 0.0)
    m = jnp.dot(a2, w3_ref[...], preferred_element_type=jnp.float32)
    m = m + b3_ref[...]
    dout = m.shape[1]
    m = m.reshape(qb, k, dout) + maskf_ref[0][:, :, None]
    o_ref[0] = _max_mid(m)


def _pair_mlp(g, posq, maskf, wp, b1, w2, b2, w3, b3, qb):
    nb, q, _ = posq.shape
    k = _MAX_NB
    dpad = g.shape[2]
    d1 = w2.shape[0]
    dh = w2.shape[1]
    dout = w3.shape[1]
    grid = (nb, q // qb)
    return pl.pallas_call(
        functools.partial(_pair_body, qb, d1),
        grid=grid,
        in_specs=[
            pl.BlockSpec((1, qb * k, dpad), lambda b, t: (b, t, 0)),
            pl.BlockSpec((1, qb, 3), lambda b, t: (b, t, 0)),
            pl.BlockSpec((1, qb, k), lambda b, t: (b, t, 0)),
            pl.BlockSpec((3, d1), lambda b, t: (0, 0)),
            pl.BlockSpec((1, d1), lambda b, t: (0, 0)),
            pl.BlockSpec((d1, dh), lambda b, t: (0, 0)),
            pl.BlockSpec((1, dh), lambda b, t: (0, 0)),
            pl.BlockSpec((dh, dout), lambda b, t: (0, 0)),
            pl.BlockSpec((1, dout), lambda b, t: (0, 0)),
        ],
        out_specs=pl.BlockSpec((1, qb, dout), lambda b, t: (b, t, 0)),
        out_shape=jax.ShapeDtypeStruct((nb, q, dout), jnp.float32),
    )(g, posq, maskf, wp, b1, w2, b2, w3, b3)


def _sa3_body(n_cloud, q, x_ref, p_ref, w0x_ref, w0p_ref, b0_ref, w1_ref,
              b1_ref, w2_ref, b2_ref, h0_ref, hb0_ref, h1_ref, hb1_ref,
              h2_ref, hb2_ref, o_ref):
    x = x_ref[...]
    p = p_ref[...]
    h = jnp.dot(x, w0x_ref[...], preferred_element_type=jnp.float32)
    h += jnp.dot(p, w0p_ref[...], preferred_element_type=jnp.float32)
    h = jnp.maximum(h + b0_ref[...], 0.0)
    h = jnp.dot(h, w1_ref[...], preferred_element_type=jnp.float32)
    h = jnp.maximum(h + b1_ref[...], 0.0)
    h = jnp.dot(h, w2_ref[...], preferred_element_type=jnp.float32)
    h = h + b2_ref[...]
    h = h.reshape(n_cloud, q, h.shape[1])
    g = _max_mid(h)
    g = jnp.dot(g, h0_ref[...], preferred_element_type=jnp.float32)
    g = jnp.maximum(g + hb0_ref[...], 0.0)
    g = jnp.dot(g, h1_ref[...], preferred_element_type=jnp.float32)
    g = jnp.maximum(g + hb1_ref[...], 0.0)
    g = jnp.dot(g, h2_ref[...], preferred_element_type=jnp.float32)
    g = g + hb2_ref[...]
    mx = jnp.max(g, axis=1, keepdims=True)
    lse = jnp.log(jnp.sum(jnp.exp(g - mx), axis=1, keepdims=True)) + mx
    o_ref[...] = g - lse


def _sa3_head(x2d, p2d, n_cloud, q, args):
    ncls = args[-1].shape[1]
    return pl.pallas_call(
        functools.partial(_sa3_body, n_cloud, q),
        out_shape=jax.ShapeDtypeStruct((n_cloud, ncls), jnp.float32),
    )(x2d, p2d, *args)


def _fold_norm(params):
    s = (1.0 + _EPS) ** 0.5
    ws, bs = [], []
    n = len(params["W"])
    for i in range(n):
        w, b = params["W"][i], params["b"][i]
        if i < n - 1:
            g, beta = params["g"][i], params["beta"][i]
            ws.append(w * (g / s)[None, :])
            bs.append(b * g / s + beta)
        else:
            ws.append(w)
            bs.append(b)
    return ws, bs


def _select_neighbors(posq, px, py, pz, r):
    d2 = ((posq[:, :, 0:1] - px[:, None, :]) ** 2
          + (posq[:, :, 1:2] - py[:, None, :]) ** 2
          + (posq[:, :, 2:3] - pz[:, None, :]) ** 2)
    score = jnp.where(d2 <= r * r, -d2, -jnp.inf)
    vals, idx = jax.lax.top_k(score, _MAX_NB)
    maskf = jnp.where(vals > -jnp.inf, 0.0, -jnp.inf).astype(jnp.float32)
    return idx.astype(jnp.int32), maskf


def _sc_gather(u_flat, idx_flat, d):
    tot = idx_flat.shape[0]
    info = plsc.get_sparse_core_info()
    nw = info.num_cores * info.num_subcores
    rows_w = tot // nw
    ch = 512
    n_chunk = rows_w // ch
    mesh = plsc.VectorSubcoreMesh(core_axis_name="c", subcore_axis_name="s")

    @functools.partial(
        pl.kernel, mesh=mesh,
        out_type=jax.ShapeDtypeStruct((tot, d), jnp.float32),
        scratch_types=[
            pltpu.VMEM((ch,), jnp.int32),
            pltpu.VMEM((ch, d), jnp.float32),
            pltpu.SemaphoreType.DMA,
        ],
    )
    def gather_k(u_hbm, idx_hbm, out_hbm, idx_v, rows_v, sem):
        wid = lax.axis_index("s") * info.num_cores + lax.axis_index("c")
        base = wid * rows_w

        def chunk(ci, carry):
            off = base + ci * ch
            pltpu.sync_copy(idx_hbm.at[pl.ds(off, ch)], idx_v)
            pltpu.async_copy(u_hbm.at[idx_v], rows_v, sem).wait()
            pltpu.sync_copy(rows_v, out_hbm.at[pl.ds(off, ch)])
            return carry

        lax.fori_loop(0, n_chunk, chunk, 0)

    return gather_k(u_flat, idx_flat)


def _gather_pairs(u, idx):
    nb, q, k = idx.shape
    n, d = u.shape[1], u.shape[2]
    gbase = jnp.arange(nb, dtype=jnp.int32)[:, None, None] * n
    flat = (idx + gbase).reshape(nb * q * k)
    out = _sc_gather(u.reshape(nb * n, d), flat, d)
    return out.reshape(nb, q * k, d)


def _sa_stage(xb, posb, px, py, pz, params, ratio, r, qb):
    nb, n_pts, fdim = xb.shape
    n_samp = int(n_pts * ratio)
    ws, bs = _fold_norm(params)
    w1, w2, w3 = ws
    b1, b2, b3 = bs
    d1 = w1.shape[1]

    sel, qx, qy, qz = _fps(px, py, pz, n_samp)
    posq = jnp.stack([qx, qy, qz], axis=-1)

    u = _compute_u(xb.reshape(nb * n_pts, fdim),
                   posb.reshape(nb * n_pts, 3),
                   w1[:fdim], w1[fdim:]).reshape(nb, n_pts, d1)
    if d1 < 128:
        u = jnp.pad(u, ((0, 0), (0, 0), (0, 128 - d1)))

    idx, maskf = _select_neighbors(posq, px, py, pz, r)
    g = _gather_pairs(u, idx)

    x_out = _pair_mlp(g, posq, maskf, w1[fdim:], b1[None, :], w2,
                      b2[None, :], w3, b3[None, :], qb)
    return x_out, posq, qx, qy, qz


def kernel(x, pos, batch, params):
    nb = batch.shape[0] // 1024
    n_pts = 1024
    xb = x.reshape(nb, n_pts, -1)
    pb = pos.reshape(nb, n_pts, 3)
    px = pb[:, :, 0]
    py = pb[:, :, 1]
    pz = pb[:, :, 2]

    x1, posq1, q1x, q1y, q1z = _sa_stage(xb, pb, px, py, pz, params["sa1"],
                                         0.5, 0.2, 128)
    x2, posq2, _, _, _ = _sa_stage(x1, posq1, q1x, q1y, q1z, params["sa2"],
                                   0.25, 0.4, 128)

    ws, bs = _fold_norm(params["sa3"])
    hw, hb = _fold_norm(params["head"])
    q2 = x2.shape[1]
    f2 = x2.shape[2]
    args = (ws[0][:f2], ws[0][f2:], bs[0][None, :], ws[1], bs[1][None, :],
            ws[2], bs[2][None, :], hw[0], hb[0][None, :], hw[1],
            hb[1][None, :], hw[2], hb[2][None, :])
    return _sa3_head(x2.reshape(nb * q2, f2), posq2.reshape(nb * q2, 3),
                     nb, q2, args)
